# baseline (speedup 1.0000x reference)
import functools

import jax
import jax.numpy as jnp
from jax import lax
from jax.experimental import pallas as pl
from jax.experimental.pallas import tpu as pltpu
from jax.experimental.pallas import tpu_sc as plsc

NC = 2
NS = 16
NW = NC * NS


def _make_gather(B, V, D, C, NBUF=4):
    b_per_w = B // NW
    n = b_per_w // C
    assert b_per_w % C == 0 and n % NBUF == 0 and n >= NBUF
    mesh = plsc.VectorSubcoreMesh(core_axis_name="c", subcore_axis_name="s")

    @functools.partial(
        pl.kernel,
        mesh=mesh,
        out_type=jax.ShapeDtypeStruct((B, D), jnp.float32),
        scratch_types=[
            pltpu.VMEM((NBUF, C), jnp.int32),
            pltpu.VMEM((NBUF, C, D), jnp.float32),
            pltpu.SemaphoreType.DMA((NBUF,)),
            pltpu.SemaphoreType.DMA((NBUF,)),
        ],
        compiler_params=pltpu.CompilerParams(use_tc_tiling_on_sc=False),
    )
    def emb(idx_hbm, table_hbm, out_hbm, idx_v, rows_v, gsem, osem):
        wid = lax.axis_index("s") * NC + lax.axis_index("c")
        base = wid * b_per_w

        def idx_sl(i):
            return idx_hbm.at[pl.ds(base + i * C, C)]

        def out_sl(i):
            return out_hbm.at[pl.ds(base + i * C, C)]

        def gather(i, s):
            pltpu.sync_copy(idx_sl(i), idx_v.at[s])
            pltpu.async_copy(table_hbm.at[idx_v.at[s]], rows_v.at[s], gsem.at[s])

        def wait_gather(s):
            pltpu.make_async_copy(
                table_hbm.at[idx_v.at[s]], rows_v.at[s], gsem.at[s]).wait()

        def wait_store(i, s):
            pltpu.make_async_copy(rows_v.at[s], out_sl(i), osem.at[s]).wait()

        gather(0, 0)
        gather(1, 1)

        @pl.loop(0, n, step=NBUF)
        def _(g):
            for b in range(NBUF):
                i = g + b
                wait_gather(b)
                pltpu.async_copy(rows_v.at[b], out_sl(i), osem.at[b])
                s = (b + 2) % NBUF

                @pl.when(i + 2 < n)
                def _():
                    @pl.when(i >= 2)
                    def _():
                        wait_store(i - 2, s)
                    gather(i + 2, s)

        for b in range(NBUF):
            i = n - NBUF + b
            wait_store(i, i % NBUF)

    return emb


def kernel(x, table):
    B0, S = x.shape
    V, D = table.shape
    H = S // 2
    g = _make_gather(B0 * H, V, D, C=400, NBUF=4)
    outA = g(x[:, :H].reshape(B0 * H).astype(jnp.int32), table)
    outB = g(x[:, H:].reshape(B0 * H).astype(jnp.int32), table)
    out = jnp.concatenate(
        [outA.reshape(B0, H, D), outB.reshape(B0, H, D)], axis=1)
    return out


# speedup vs baseline: 1.0879x; 1.0879x over previous
import functools

import jax
import jax.numpy as jnp
from jax import lax
from jax.experimental import pallas as pl
from jax.experimental.pallas import tpu as pltpu
from jax.experimental.pallas import tpu_sc as plsc

NC = 2
NS = 16
NW = NC * NS


def _make_gather(B, V, D, C, NBUF=4):
    b_per_w = B // NW
    n = b_per_w // C
    assert b_per_w % C == 0 and n % NBUF == 0 and n >= NBUF
    mesh = plsc.VectorSubcoreMesh(core_axis_name="c", subcore_axis_name="s")

    @functools.partial(
        pl.kernel,
        mesh=mesh,
        out_type=jax.ShapeDtypeStruct((B, D), jnp.float32),
        scratch_types=[
            pltpu.VMEM((b_per_w,), jnp.int32),
            pltpu.VMEM((NBUF, C, D), jnp.float32),
            pltpu.SemaphoreType.DMA((NBUF,)),
            pltpu.SemaphoreType.DMA((NBUF,)),
        ],
        compiler_params=pltpu.CompilerParams(use_tc_tiling_on_sc=False),
    )
    def emb(idx_hbm, table_hbm, out_hbm, idx_v, rows_v, gsem, osem):
        wid = lax.axis_index("s") * NC + lax.axis_index("c")
        base = wid * b_per_w

        pltpu.sync_copy(idx_hbm.at[pl.ds(base, b_per_w)], idx_v)

        def idx_sl(i):
            return idx_v.at[pl.ds(i * C, C)]

        def out_sl(i):
            return out_hbm.at[pl.ds(base + i * C, C)]

        def gather(i, s):
            pltpu.async_copy(table_hbm.at[idx_sl(i)], rows_v.at[s], gsem.at[s])

        def wait_gather(i, s):
            pltpu.make_async_copy(
                table_hbm.at[idx_sl(i)], rows_v.at[s], gsem.at[s]).wait()

        def wait_store(i, s):
            pltpu.make_async_copy(rows_v.at[s], out_sl(i), osem.at[s]).wait()

        gather(0, 0)
        gather(1, 1)

        @pl.loop(0, n, step=NBUF)
        def _(g):
            for b in range(NBUF):
                i = g + b
                wait_gather(i, b)
                pltpu.async_copy(rows_v.at[b], out_sl(i), osem.at[b])
                s = (b + 2) % NBUF

                @pl.when(i + 2 < n)
                def _():
                    @pl.when(i >= 2)
                    def _():
                        wait_store(i - 2, s)
                    gather(i + 2, s)

        for b in range(NBUF):
            i = n - NBUF + b
            wait_store(i, i % NBUF)

    return emb


def kernel(x, table):
    B0, S = x.shape
    V, D = table.shape
    B = B0 * S
    xf = x.reshape(B).astype(jnp.int32)
    out = _make_gather(B, V, D, C=320, NBUF=4)(xf, table)
    return out.reshape(B0, S, D)


# speedup vs baseline: 1.0892x; 1.0012x over previous
import functools

import jax
import jax.numpy as jnp
from jax import lax
from jax.experimental import pallas as pl
from jax.experimental.pallas import tpu as pltpu
from jax.experimental.pallas import tpu_sc as plsc

NC = 2
NS = 16
NW = NC * NS


def _make_gather(B, V, D, C, NBUF=4):
    b_per_w = B // NW
    n = b_per_w // C
    assert b_per_w % C == 0 and n % NBUF == 0 and n >= NBUF
    mesh = plsc.VectorSubcoreMesh(core_axis_name="c", subcore_axis_name="s")

    @functools.partial(
        pl.kernel,
        mesh=mesh,
        out_type=jax.ShapeDtypeStruct((B, D), jnp.float32),
        scratch_types=[
            pltpu.VMEM((NBUF, C), jnp.int32),
            pltpu.VMEM((NBUF, C, D), jnp.float32),
            pltpu.SemaphoreType.DMA((NBUF,)),
            pltpu.SemaphoreType.DMA((NBUF,)),
        ],
        compiler_params=pltpu.CompilerParams(use_tc_tiling_on_sc=False),
    )
    def emb(idx_hbm, table_hbm, out_hbm, idx_v, rows_v, gsem, osem):
        wid = lax.axis_index("s") * NC + lax.axis_index("c")
        base = wid * b_per_w

        def idx_sl(i):
            return idx_hbm.at[pl.ds(base + i * C, C)]

        def out_sl(i):
            return out_hbm.at[pl.ds(base + i * C, C)]

        def gather(i, s):
            pltpu.sync_copy(idx_sl(i), idx_v.at[s])
            pltpu.async_copy(table_hbm.at[idx_v.at[s]], rows_v.at[s], gsem.at[s])

        def wait_gather(s):
            pltpu.make_async_copy(
                table_hbm.at[idx_v.at[s]], rows_v.at[s], gsem.at[s]).wait()

        def wait_store(i, s):
            pltpu.make_async_copy(rows_v.at[s], out_sl(i), osem.at[s]).wait()

        gather(0, 0)
        gather(1, 1)

        @pl.loop(0, n, step=NBUF)
        def _(g):
            for b in range(NBUF):
                i = g + b
                wait_gather(b)
                pltpu.async_copy(rows_v.at[b], out_sl(i), osem.at[b])
                s = (b + 2) % NBUF
                @pl.when(i + 2 < n)
                def _():
                    @pl.when(i >= 2)
                    def _():
                        wait_store(i - 2, s)
                    gather(i + 2, s)

        for b in range(NBUF):
            i = n - NBUF + b
            wait_store(i, i % NBUF)

    return emb


def kernel(x, table):
    B0, S = x.shape
    V, D = table.shape
    B = B0 * S
    xf = x.reshape(B).astype(jnp.int32)
    out = _make_gather(B, V, D, C=400, NBUF=4)(xf, table)
    return out.reshape(B0, S, D)


# speedup vs baseline: 1.0905x; 1.0012x over previous
import functools

import jax
import jax.numpy as jnp
from jax import lax
from jax.experimental import pallas as pl
from jax.experimental.pallas import tpu as pltpu
from jax.experimental.pallas import tpu_sc as plsc

NC = 2
NS = 16
NW = NC * NS


def _make_gather(B, V, D, C, NBUF=4):
    b_per_w = B // NW
    n = b_per_w // C
    assert b_per_w % C == 0 and n % NBUF == 0 and n >= NBUF
    mesh = plsc.VectorSubcoreMesh(core_axis_name="c", subcore_axis_name="s")

    @functools.partial(
        pl.kernel,
        mesh=mesh,
        out_type=jax.ShapeDtypeStruct((B, D), jnp.float32),
        scratch_types=[
            pltpu.VMEM((NBUF, C), jnp.int32),
            pltpu.VMEM((NBUF, C, D), jnp.float32),
            pltpu.SemaphoreType.DMA((NBUF,)),
            pltpu.SemaphoreType.DMA((NBUF,)),
        ],
        compiler_params=pltpu.CompilerParams(use_tc_tiling_on_sc=False),
    )
    def emb(idx_hbm, table_hbm, out_hbm, idx_v, rows_v, gsem, osem):
        wid = lax.axis_index("s") * NC + lax.axis_index("c")
        base = wid * b_per_w

        def idx_sl(i):
            return idx_hbm.at[pl.ds(base + i * C, C)]

        def out_sl(i):
            return out_hbm.at[pl.ds(base + i * C, C)]

        def gather(i, s):
            pltpu.sync_copy(idx_sl(i), idx_v.at[s])
            pltpu.async_copy(table_hbm.at[idx_v.at[s]], rows_v.at[s], gsem.at[s])

        def wait_gather(s):
            pltpu.make_async_copy(
                table_hbm.at[idx_v.at[s]], rows_v.at[s], gsem.at[s]).wait()

        def wait_store(i, s):
            pltpu.make_async_copy(rows_v.at[s], out_sl(i), osem.at[s]).wait()

        gather(0, 0)
        gather(1, 1)
        gather(2, 2)

        @pl.loop(0, n, step=NBUF)
        def _(g):
            for b in range(NBUF):
                i = g + b
                wait_gather(b)
                pltpu.async_copy(rows_v.at[b], out_sl(i), osem.at[b])
                s = (b + 3) % NBUF
                @pl.when(i + 3 < n)
                def _():
                    @pl.when(i >= 2)
                    def _():
                        wait_store(i - 2, s)
                    gather(i + 3, s)

        for b in range(NBUF):
            i = n - NBUF + b
            wait_store(i, i % NBUF)

    return emb


def kernel(x, table):
    B0, S = x.shape
    V, D = table.shape
    B = B0 * S
    xf = x.reshape(B).astype(jnp.int32)
    out = _make_gather(B, V, D, C=320, NBUF=5)(xf, table)
    return out.reshape(B0, S, D)
